# baseline (speedup 1.0000x reference)
import functools

import jax
import jax.numpy as jnp
from jax import lax
from jax.experimental import pallas as pl
from jax.experimental.pallas import tpu as pltpu
from jax.experimental.pallas import tpu_sc as plsc

N = 10000
E = 320000
D = 128
NE = 4
NSTEPS = 3
G = 128
NCLS = 10

NC = 2
NS = 16
CE = 128
EPT = 10240
CH = EPT // CE
NSTAGE = 4
CHH = CH // NSTAGE
PAD = EPT - E // (NC * NS)
ACCN = N + 256
RPT = 624

_DN = (((1,), (1,)), ((), ()))
_PREC = jax.lax.Precision.DEFAULT

NB = 5
BN = N // NB


def _matmul(x, w):
    return jax.lax.dot_general(x, w, _DN, precision=_PREC,
                               preferred_element_type=jnp.float32)


def _gru(a, h, wih, whh, bih, bhh):
    gi = _matmul(a, wih) + bih
    gh = _matmul(h, whh) + bhh
    r = jax.nn.sigmoid(gi[:, :D] + gh[:, :D])
    z = jax.nn.sigmoid(gi[:, D:2 * D] + gh[:, D:2 * D])
    n = jnp.tanh(gi[:, 2 * D:] + r * gh[:, 2 * D:])
    return (1.0 - z) * n + z * h


def _mm_body(h_ref, w_ref, b_ref, m_ref):
    h = h_ref[...]
    for e in range(NE):
        m_ref[e] = _matmul(h, w_ref[e]) + b_ref[e][None, :]


_mm_call = pl.pallas_call(
    _mm_body,
    grid=(NB,),
    in_specs=[
        pl.BlockSpec((BN, D), lambda i: (i, 0)),
        pl.BlockSpec((NE, D, D), lambda i: (0, 0, 0)),
        pl.BlockSpec((NE, D), lambda i: (0, 0)),
    ],
    out_specs=pl.BlockSpec((NE, BN, D), lambda i: (0, i, 0)),
    out_shape=jax.ShapeDtypeStruct((NE, N, D), jnp.float32),
)


def _gru_body(ap_ref, h_ref, wih_ref, whh_ref, bih_ref, bhh_ref, w_ref, b_ref,
              h_out, m_out, *, relu, emit_m):
    a = ap_ref[0] + ap_ref[1]
    hn = _gru(a, h_ref[...], wih_ref[...], whh_ref[...], bih_ref[...],
              bhh_ref[...])
    if relu:
        hn = jnp.maximum(hn, 0.0)
    h_out[...] = hn
    if emit_m:
        for e in range(NE):
            m_out[e] = _matmul(hn, w_ref[e]) + b_ref[e][None, :]


def _make_gru_call(relu, emit_m):
    out_shape = [jax.ShapeDtypeStruct((N, D), jnp.float32)]
    out_specs = [pl.BlockSpec((BN, D), lambda i: (i, 0))]
    if emit_m:
        out_shape.append(jax.ShapeDtypeStruct((NE, N, D), jnp.float32))
        out_specs.append(pl.BlockSpec((NE, BN, D), lambda i: (0, i, 0)))

    def body(ap, h, wih, whh, bih, bhh, w, b, h_out, *maybe_m):
        _gru_body(ap, h, wih, whh, bih, bhh, w, b, h_out,
                  maybe_m[0] if emit_m else None, relu=relu, emit_m=emit_m)

    return pl.pallas_call(
        body,
        grid=(NB,),
        in_specs=[
            pl.BlockSpec((NC, BN, D), lambda i: (0, i, 0)),
            pl.BlockSpec((BN, D), lambda i: (i, 0)),
            pl.BlockSpec((3 * D, D), lambda i: (0, 0)),
            pl.BlockSpec((3 * D, D), lambda i: (0, 0)),
            pl.BlockSpec((1, 3 * D), lambda i: (0, 0)),
            pl.BlockSpec((1, 3 * D), lambda i: (0, 0)),
            pl.BlockSpec((NE, D, D), lambda i: (0, 0, 0)),
            pl.BlockSpec((NE, D), lambda i: (0, 0)),
        ],
        out_specs=out_specs,
        out_shape=out_shape,
    )


_gru_m_call = _make_gru_call(relu=False, emit_m=True)
_gru_m_relu_call = _make_gru_call(relu=True, emit_m=True)
_gru_last_call = _make_gru_call(relu=True, emit_m=False)


def _edge_body(m_hbm, g_hbm, d_hbm, z_hbm, out_hbm, gv, dv, rows, acc, sem):
    c = lax.axis_index("c")
    s = lax.axis_index("s")
    wid = c * NS + s
    tail0 = NS * RPT
    pltpu.sync_copy(z_hbm.at[pl.ds(s * RPT, RPT)],
                    acc.at[pl.ds(s * RPT, RPT)])

    @pl.when(s == NS - 1)
    def _():
        pltpu.sync_copy(z_hbm.at[pl.ds(tail0, ACCN - tail0)],
                        acc.at[pl.ds(tail0, ACCN - tail0)])

    plsc.subcore_barrier()

    rows0, rows1 = rows
    sem0, sem1 = sem
    for stage in range(NSTAGE):
        pltpu.sync_copy(g_hbm.at[wid, stage], gv)
        pltpu.sync_copy(d_hbm.at[wid, stage], dv)
        pltpu.async_copy(m_hbm.at[gv.at[0]], rows0, sem0)

        def chunk(i, carry):
            j = 2 * i
            pltpu.async_copy(m_hbm.at[gv.at[j + 1]], rows1, sem1)
            pltpu.make_async_copy(m_hbm.at[gv.at[j]], rows0, sem0).wait()
            pltpu.sync_copy(rows0, acc.at[dv.at[j]], add=True)

            @pl.when(j + 2 < CHH)
            def _():
                pltpu.async_copy(m_hbm.at[gv.at[j + 2]], rows0, sem0)

            pltpu.make_async_copy(m_hbm.at[gv.at[j + 1]], rows1, sem1).wait()
            pltpu.sync_copy(rows1, acc.at[dv.at[j + 1]], add=True)
            return carry

        lax.fori_loop(0, CHH // 2, chunk, 0)
    plsc.subcore_barrier()
    pltpu.sync_copy(acc.at[pl.ds(s * RPT, RPT)],
                    out_hbm.at[c, pl.ds(s * RPT, RPT)])

    @pl.when(s == NS - 1)
    def _():
        pltpu.sync_copy(acc.at[pl.ds(tail0, N - tail0)],
                        out_hbm.at[c, pl.ds(tail0, N - tail0)])


_edge_call = functools.partial(
    pl.kernel,
    out_type=jax.ShapeDtypeStruct((NC, N, D), jnp.float32),
    mesh=plsc.VectorSubcoreMesh(core_axis_name="c", subcore_axis_name="s"),
    scratch_types=[
        pltpu.VMEM((CHH, CE), jnp.int32),
        pltpu.VMEM((CHH, CE), jnp.int32),
        (pltpu.VMEM((CE, D), jnp.float32), pltpu.VMEM((CE, D), jnp.float32)),
        pltpu.VMEM_SHARED((ACCN, D), jnp.float32),
        (pltpu.SemaphoreType.DMA, pltpu.SemaphoreType.DMA),
    ],
)(_edge_body)


def _pool_body(h_ref, gid_ref, gw_ref, gb_ref, fw_ref, fb_ref, out_ref):
    h = h_ref[...]
    gate = jnp.sum(h * gw_ref[...], axis=1, keepdims=True) + gb_ref[0, 0]
    onehot_b = gid_ref[...] == jax.lax.broadcasted_iota(jnp.int32, (1, G), 1)
    one = onehot_b.astype(jnp.float32)
    gmax = jnp.max(jnp.where(onehot_b, gate, -1e30), axis=0, keepdims=True)
    ge = jnp.exp(gate - jnp.sum(one * gmax, axis=1, keepdims=True))
    denom = jnp.sum(one * ge, axis=0, keepdims=True)
    denom_n = jnp.sum(one * denom, axis=1, keepdims=True)
    wh = (ge / jnp.maximum(denom_n, 1e-12)) * h
    hg = jax.lax.dot_general(one, wh, (((0,), (0,)), ((), ())),
                             precision=_PREC,
                             preferred_element_type=jnp.float32)
    out_ref[...] = _matmul(hg, fw_ref[...]) + fb_ref[...]


_pool_call = pl.pallas_call(
    _pool_body,
    in_specs=[
        pl.BlockSpec((N, D), lambda: (0, 0)),
        pl.BlockSpec((N, 1), lambda: (0, 0)),
        pl.BlockSpec((1, D), lambda: (0, 0)),
        pl.BlockSpec((1, 1), lambda: (0, 0)),
        pl.BlockSpec((NCLS, D), lambda: (0, 0)),
        pl.BlockSpec((1, NCLS), lambda: (0, 0)),
    ],
    out_specs=pl.BlockSpec((G, NCLS), lambda: (0, 0)),
    out_shape=jax.ShapeDtypeStruct((G, NCLS), jnp.float32),
)


def kernel(feat, edge_index, etypes, graph_ids, W1, b1, gru1_wih, gru1_whh,
           gru1_bih, gru1_bhh, W2, b2, gru2_wih, gru2_whh, gru2_bih, gru2_bhh,
           gate_w, gate_b, fc_w, fc_b):
    src = edge_index[0].astype(jnp.int32)
    dst = edge_index[1].astype(jnp.int32)
    nw = NC * NS
    gflat = (etypes.astype(jnp.int32) * N + src).reshape(nw, E // nw)
    dflat = dst.reshape(nw, E // nw)
    gpad = jnp.zeros((nw, PAD), jnp.int32)
    dpad = (N + 8 * jnp.arange(nw, dtype=jnp.int32)[:, None]
            + (jnp.arange(PAD, dtype=jnp.int32) % 8)[None, :])
    gidx = jnp.concatenate([gflat, gpad], 1).reshape(nw, NSTAGE, CHH, CE)
    didx = jnp.concatenate([dflat, dpad], 1).reshape(nw, NSTAGE, CHH, CE)
    zeros = jnp.zeros((ACCN, D), jnp.float32)
    gid2 = graph_ids.astype(jnp.int32).reshape(N, 1)
    bih1 = gru1_bih.reshape(1, 3 * D)
    bhh1 = gru1_bhh.reshape(1, 3 * D)
    bih2 = gru2_bih.reshape(1, 3 * D)
    bhh2 = gru2_bhh.reshape(1, 3 * D)
    gb2 = gate_b.reshape(1, 1)
    fb2 = fc_b.reshape(1, NCLS)

    h = feat
    m = _mm_call(h, W1, b1)
    for layer in range(2):
        wih, whh, bih, bhh = ((gru1_wih, gru1_whh, bih1, bhh1) if layer == 0
                              else (gru2_wih, gru2_whh, bih2, bhh2))
        for step in range(NSTEPS):
            ap = _edge_call(m.reshape(NE * N, D), gidx, didx, zeros)
            last = layer == 1 and step == NSTEPS - 1
            boundary = layer == 0 and step == NSTEPS - 1
            if last:
                (h,) = _gru_last_call(ap, h, wih, whh, bih, bhh, W2, b2)
            elif boundary:
                h, m = _gru_m_relu_call(ap, h, wih, whh, bih, bhh, W2, b2)
            else:
                Wc, bc = (W1, b1) if layer == 0 else (W2, b2)
                h, m = _gru_m_call(ap, h, wih, whh, bih, bhh, Wc, bc)
    return _pool_call(h, gid2, gate_w, gb2, fc_w, fb2)


# speedup vs baseline: 2.9480x; 2.9480x over previous
import functools

import jax
import jax.numpy as jnp
from jax import lax
from jax.experimental import pallas as pl
from jax.experimental.pallas import tpu as pltpu
from jax.experimental.pallas import tpu_sc as plsc

N = 10000
E = 320000
D = 128
NE = 4
NSTEPS = 3
G = 128
NCLS = 10

NC = 2
NS = 16
CE = 128
EPT = 10240
CH = EPT // CE
NSTAGE = 4
CHH = CH // NSTAGE
PAD = EPT - E // (NC * NS)
ACCN = N + 256
RPT = 624

_DN = (((1,), (1,)), ((), ()))
_PREC = jax.lax.Precision.DEFAULT

NB = 5
BN = N // NB


def _matmul(x, w):
    return jax.lax.dot_general(x, w, _DN, precision=_PREC,
                               preferred_element_type=jnp.float32)


def _gru(a, h, wih, whh, bih, bhh):
    gi = _matmul(a, wih) + bih
    gh = _matmul(h, whh) + bhh
    r = jax.nn.sigmoid(gi[:, :D] + gh[:, :D])
    z = jax.nn.sigmoid(gi[:, D:2 * D] + gh[:, D:2 * D])
    n = jnp.tanh(gi[:, 2 * D:] + r * gh[:, 2 * D:])
    return (1.0 - z) * n + z * h


def _mm_body(h_ref, w_ref, b_ref, m_ref):
    h = h_ref[...]
    for e in range(NE):
        m_ref[e] = _matmul(h, w_ref[e]) + b_ref[e][None, :]


_mm_call = pl.pallas_call(
    _mm_body,
    grid=(NB,),
    in_specs=[
        pl.BlockSpec((BN, D), lambda i: (i, 0)),
        pl.BlockSpec((NE, D, D), lambda i: (0, 0, 0)),
        pl.BlockSpec((NE, D), lambda i: (0, 0)),
    ],
    out_specs=pl.BlockSpec((NE, BN, D), lambda i: (0, i, 0)),
    out_shape=jax.ShapeDtypeStruct((NE, N, D), jnp.float32),
)


def _gru_body(ap_ref, h_ref, wih_ref, whh_ref, bih_ref, bhh_ref, w_ref, b_ref,
              h_out, m_out, *, relu, emit_m):
    a = ap_ref[0] + ap_ref[1]
    hn = _gru(a, h_ref[...], wih_ref[...], whh_ref[...], bih_ref[...],
              bhh_ref[...])
    if relu:
        hn = jnp.maximum(hn, 0.0)
    h_out[...] = hn
    if emit_m:
        for e in range(NE):
            m_out[e] = _matmul(hn, w_ref[e]) + b_ref[e][None, :]


def _make_gru_call(relu, emit_m):
    out_shape = [jax.ShapeDtypeStruct((N, D), jnp.float32)]
    out_specs = [pl.BlockSpec((BN, D), lambda i: (i, 0))]
    if emit_m:
        out_shape.append(jax.ShapeDtypeStruct((NE, N, D), jnp.float32))
        out_specs.append(pl.BlockSpec((NE, BN, D), lambda i: (0, i, 0)))

    def body(ap, h, wih, whh, bih, bhh, w, b, h_out, *maybe_m):
        _gru_body(ap, h, wih, whh, bih, bhh, w, b, h_out,
                  maybe_m[0] if emit_m else None, relu=relu, emit_m=emit_m)

    return pl.pallas_call(
        body,
        grid=(NB,),
        in_specs=[
            pl.BlockSpec((NC, BN, D), lambda i: (0, i, 0)),
            pl.BlockSpec((BN, D), lambda i: (i, 0)),
            pl.BlockSpec((3 * D, D), lambda i: (0, 0)),
            pl.BlockSpec((3 * D, D), lambda i: (0, 0)),
            pl.BlockSpec((1, 3 * D), lambda i: (0, 0)),
            pl.BlockSpec((1, 3 * D), lambda i: (0, 0)),
            pl.BlockSpec((NE, D, D), lambda i: (0, 0, 0)),
            pl.BlockSpec((NE, D), lambda i: (0, 0)),
        ],
        out_specs=out_specs,
        out_shape=out_shape,
    )


_gru_m_call = _make_gru_call(relu=False, emit_m=True)
_gru_m_relu_call = _make_gru_call(relu=True, emit_m=True)
_gru_last_call = _make_gru_call(relu=True, emit_m=False)


def _edge_body(m_hbm, g_hbm, d_hbm, z_hbm, out_hbm, gv, dv, rows, acc, sem):
    c = lax.axis_index("c")
    s = lax.axis_index("s")
    wid = c * NS + s
    tail0 = NS * RPT
    pltpu.sync_copy(z_hbm.at[pl.ds(s * RPT, RPT)],
                    acc.at[pl.ds(s * RPT, RPT)])

    @pl.when(s == NS - 1)
    def _():
        pltpu.sync_copy(z_hbm.at[pl.ds(tail0, ACCN - tail0)],
                        acc.at[pl.ds(tail0, ACCN - tail0)])

    plsc.subcore_barrier()

    rows0, rows1 = rows
    sem0, sem1 = sem
    for stage in range(NSTAGE):
        pltpu.sync_copy(g_hbm.at[wid, stage], gv)
        pltpu.sync_copy(d_hbm.at[wid, stage], dv)
        pltpu.async_copy(m_hbm.at[gv.at[0]], rows0, sem0)

        def chunk(i, carry):
            j = 2 * i
            pltpu.async_copy(m_hbm.at[gv.at[j + 1]], rows1, sem1)
            pltpu.make_async_copy(m_hbm.at[gv.at[j]], rows0, sem0).wait()
            pltpu.sync_copy(rows0, acc.at[dv.at[j]], add=True)

            @pl.when(j + 2 < CHH)
            def _():
                pltpu.async_copy(m_hbm.at[gv.at[j + 2]], rows0, sem0)

            pltpu.make_async_copy(m_hbm.at[gv.at[j + 1]], rows1, sem1).wait()
            pltpu.sync_copy(rows1, acc.at[dv.at[j + 1]], add=True)
            return carry

        lax.fori_loop(0, CHH // 2, chunk, 0)
    plsc.subcore_barrier()
    pltpu.sync_copy(acc.at[pl.ds(s * RPT, RPT)],
                    out_hbm.at[c, pl.ds(s * RPT, RPT)])

    @pl.when(s == NS - 1)
    def _():
        pltpu.sync_copy(acc.at[pl.ds(tail0, N - tail0)],
                        out_hbm.at[c, pl.ds(tail0, N - tail0)])


_edge_call = functools.partial(
    pl.kernel,
    out_type=jax.ShapeDtypeStruct((NC, N, D), jnp.float32),
    mesh=plsc.VectorSubcoreMesh(core_axis_name="c", subcore_axis_name="s"),
    scratch_types=[
        pltpu.VMEM((CHH, CE), jnp.int32),
        pltpu.VMEM((CHH, CE), jnp.int32),
        (pltpu.VMEM((CE, D), jnp.float32), pltpu.VMEM((CE, D), jnp.float32)),
        pltpu.VMEM_SHARED((ACCN, D), jnp.float32),
        (pltpu.SemaphoreType.DMA, pltpu.SemaphoreType.DMA),
    ],
)(_edge_body)


def _pool_body(h_ref, gid_ref, gw_ref, gb_ref, fw_ref, fb_ref, out_ref):
    h = h_ref[...]
    gate = jnp.sum(h * gw_ref[...], axis=1, keepdims=True) + gb_ref[0, 0]
    onehot_b = gid_ref[...] == jax.lax.broadcasted_iota(jnp.int32, (1, G), 1)
    one = onehot_b.astype(jnp.float32)
    gmax = jnp.max(jnp.where(onehot_b, gate, -1e30), axis=0, keepdims=True)
    ge = jnp.exp(gate - jnp.sum(one * gmax, axis=1, keepdims=True))
    denom = jnp.sum(one * ge, axis=0, keepdims=True)
    denom_n = jnp.sum(one * denom, axis=1, keepdims=True)
    wh = (ge / jnp.maximum(denom_n, 1e-12)) * h
    hg = jax.lax.dot_general(one, wh, (((0,), (0,)), ((), ())),
                             precision=_PREC,
                             preferred_element_type=jnp.float32)
    out_ref[...] = _matmul(hg, fw_ref[...]) + fb_ref[...]


_pool_call = pl.pallas_call(
    _pool_body,
    in_specs=[
        pl.BlockSpec((N, D), lambda: (0, 0)),
        pl.BlockSpec((N, 1), lambda: (0, 0)),
        pl.BlockSpec((1, D), lambda: (0, 0)),
        pl.BlockSpec((1, 1), lambda: (0, 0)),
        pl.BlockSpec((NCLS, D), lambda: (0, 0)),
        pl.BlockSpec((1, NCLS), lambda: (0, 0)),
    ],
    out_specs=pl.BlockSpec((G, NCLS), lambda: (0, 0)),
    out_shape=jax.ShapeDtypeStruct((G, NCLS), jnp.float32),
)


def kernel(feat, edge_index, etypes, graph_ids, W1, b1, gru1_wih, gru1_whh,
           gru1_bih, gru1_bhh, W2, b2, gru2_wih, gru2_whh, gru2_bih, gru2_bhh,
           gate_w, gate_b, fc_w, fc_b):
    src = edge_index[0].astype(jnp.int32)
    dst = edge_index[1].astype(jnp.int32)
    nw = NC * NS
    gflat = (etypes.astype(jnp.int32) * N + src).reshape(nw, E // nw)
    dflat = dst.reshape(nw, E // nw)
    gpad = jnp.broadcast_to((jnp.arange(PAD, dtype=jnp.int32) * 167)
                            % (NE * N), (nw, PAD))
    dpad = (N + 8 * jnp.arange(nw, dtype=jnp.int32)[:, None]
            + (jnp.arange(PAD, dtype=jnp.int32) % 8)[None, :])
    gidx = jnp.concatenate([gflat, gpad], 1).reshape(nw, NSTAGE, CHH, CE)
    didx = jnp.concatenate([dflat, dpad], 1).reshape(nw, NSTAGE, CHH, CE)
    zeros = jnp.zeros((ACCN, D), jnp.float32)
    gid2 = graph_ids.astype(jnp.int32).reshape(N, 1)
    bih1 = gru1_bih.reshape(1, 3 * D)
    bhh1 = gru1_bhh.reshape(1, 3 * D)
    bih2 = gru2_bih.reshape(1, 3 * D)
    bhh2 = gru2_bhh.reshape(1, 3 * D)
    gb2 = gate_b.reshape(1, 1)
    fb2 = fc_b.reshape(1, NCLS)

    h = feat
    m = _mm_call(h, W1, b1)
    for layer in range(2):
        wih, whh, bih, bhh = ((gru1_wih, gru1_whh, bih1, bhh1) if layer == 0
                              else (gru2_wih, gru2_whh, bih2, bhh2))
        for step in range(NSTEPS):
            ap = _edge_call(m.reshape(NE * N, D), gidx, didx, zeros)
            last = layer == 1 and step == NSTEPS - 1
            boundary = layer == 0 and step == NSTEPS - 1
            if last:
                (h,) = _gru_last_call(ap, h, wih, whh, bih, bhh, W2, b2)
            elif boundary:
                h, m = _gru_m_relu_call(ap, h, wih, whh, bih, bhh, W2, b2)
            else:
                Wc, bc = (W1, b1) if layer == 0 else (W2, b2)
                h, m = _gru_m_call(ap, h, wih, whh, bih, bhh, Wc, bc)
    return _pool_call(h, gid2, gate_w, gb2, fc_w, fb2)


# speedup vs baseline: 3.0356x; 1.0297x over previous
import functools

import jax
import jax.numpy as jnp
from jax import lax
from jax.experimental import pallas as pl
from jax.experimental.pallas import tpu as pltpu
from jax.experimental.pallas import tpu_sc as plsc

N = 10000
E = 320000
D = 128
NE = 4
NSTEPS = 3
G = 128
NCLS = 10

NC = 2
NS = 16
CE = 100
CH = E // (NC * NS * CE)
NSTAGE = 2
CHH = CH // NSTAGE
ACCN = N
RPT = 624

_DN = (((1,), (1,)), ((), ()))
_PREC = jax.lax.Precision.DEFAULT

NB = 5
BN = N // NB


def _matmul(x, w):
    return jax.lax.dot_general(x, w, _DN, precision=_PREC,
                               preferred_element_type=jnp.float32)


def _gru(a, h, wih, whh, bih, bhh):
    gi = _matmul(a, wih) + bih
    gh = _matmul(h, whh) + bhh
    r = jax.nn.sigmoid(gi[:, :D] + gh[:, :D])
    z = jax.nn.sigmoid(gi[:, D:2 * D] + gh[:, D:2 * D])
    n = jnp.tanh(gi[:, 2 * D:] + r * gh[:, 2 * D:])
    return (1.0 - z) * n + z * h


def _mm_body(h_ref, w_ref, b_ref, m_ref):
    h = h_ref[...]
    for e in range(NE):
        m_ref[e] = _matmul(h, w_ref[e]) + b_ref[e][None, :]


_mm_call = pl.pallas_call(
    _mm_body,
    grid=(NB,),
    in_specs=[
        pl.BlockSpec((BN, D), lambda i: (i, 0)),
        pl.BlockSpec((NE, D, D), lambda i: (0, 0, 0)),
        pl.BlockSpec((NE, D), lambda i: (0, 0)),
    ],
    out_specs=pl.BlockSpec((NE, BN, D), lambda i: (0, i, 0)),
    out_shape=jax.ShapeDtypeStruct((NE, N, D), jnp.float32),
)


def _gru_body(ap_ref, h_ref, wih_ref, whh_ref, bih_ref, bhh_ref, w_ref, b_ref,
              h_out, m_out, *, relu, emit_m):
    a = ap_ref[0] + ap_ref[1]
    hn = _gru(a, h_ref[...], wih_ref[...], whh_ref[...], bih_ref[...],
              bhh_ref[...])
    if relu:
        hn = jnp.maximum(hn, 0.0)
    h_out[...] = hn
    if emit_m:
        for e in range(NE):
            m_out[e] = _matmul(hn, w_ref[e]) + b_ref[e][None, :]


def _make_gru_call(relu, emit_m):
    out_shape = [jax.ShapeDtypeStruct((N, D), jnp.float32)]
    out_specs = [pl.BlockSpec((BN, D), lambda i: (i, 0))]
    if emit_m:
        out_shape.append(jax.ShapeDtypeStruct((NE, N, D), jnp.float32))
        out_specs.append(pl.BlockSpec((NE, BN, D), lambda i: (0, i, 0)))

    def body(ap, h, wih, whh, bih, bhh, w, b, h_out, *maybe_m):
        _gru_body(ap, h, wih, whh, bih, bhh, w, b, h_out,
                  maybe_m[0] if emit_m else None, relu=relu, emit_m=emit_m)

    return pl.pallas_call(
        body,
        grid=(NB,),
        in_specs=[
            pl.BlockSpec((NC, BN, D), lambda i: (0, i, 0)),
            pl.BlockSpec((BN, D), lambda i: (i, 0)),
            pl.BlockSpec((3 * D, D), lambda i: (0, 0)),
            pl.BlockSpec((3 * D, D), lambda i: (0, 0)),
            pl.BlockSpec((1, 3 * D), lambda i: (0, 0)),
            pl.BlockSpec((1, 3 * D), lambda i: (0, 0)),
            pl.BlockSpec((NE, D, D), lambda i: (0, 0, 0)),
            pl.BlockSpec((NE, D), lambda i: (0, 0)),
        ],
        out_specs=out_specs,
        out_shape=out_shape,
    )


_gru_m_call = _make_gru_call(relu=False, emit_m=True)
_gru_m_relu_call = _make_gru_call(relu=True, emit_m=True)
_gru_last_call = _make_gru_call(relu=True, emit_m=False)


def _edge_body(m_hbm, g_hbm, d_hbm, z_hbm, out_hbm, gv, dv, rows, acc, sem):
    c = lax.axis_index("c")
    s = lax.axis_index("s")
    wid = c * NS + s
    tail0 = NS * RPT
    pltpu.sync_copy(z_hbm.at[pl.ds(s * RPT, RPT)],
                    acc.at[pl.ds(s * RPT, RPT)])

    @pl.when(s == NS - 1)
    def _():
        pltpu.sync_copy(z_hbm.at[pl.ds(tail0, ACCN - tail0)],
                        acc.at[pl.ds(tail0, ACCN - tail0)])

    plsc.subcore_barrier()

    rows0, rows1 = rows
    sem0, sem1 = sem
    for stage in range(NSTAGE):
        pltpu.sync_copy(g_hbm.at[wid, stage], gv)
        pltpu.sync_copy(d_hbm.at[wid, stage], dv)
        pltpu.async_copy(m_hbm.at[gv.at[0]], rows0, sem0)

        def chunk(i, carry):
            j = 2 * i
            pltpu.async_copy(m_hbm.at[gv.at[j + 1]], rows1, sem1)
            pltpu.make_async_copy(m_hbm.at[gv.at[j]], rows0, sem0).wait()
            pltpu.sync_copy(rows0, acc.at[dv.at[j]], add=True)

            @pl.when(j + 2 < CHH)
            def _():
                pltpu.async_copy(m_hbm.at[gv.at[j + 2]], rows0, sem0)

            pltpu.make_async_copy(m_hbm.at[gv.at[j + 1]], rows1, sem1).wait()
            pltpu.sync_copy(rows1, acc.at[dv.at[j + 1]], add=True)
            return carry

        lax.fori_loop(0, CHH // 2, chunk, 0)
    plsc.subcore_barrier()
    pltpu.sync_copy(acc.at[pl.ds(s * RPT, RPT)],
                    out_hbm.at[c, pl.ds(s * RPT, RPT)])

    @pl.when(s == NS - 1)
    def _():
        pltpu.sync_copy(acc.at[pl.ds(tail0, N - tail0)],
                        out_hbm.at[c, pl.ds(tail0, N - tail0)])


_edge_call = functools.partial(
    pl.kernel,
    out_type=jax.ShapeDtypeStruct((NC, N, D), jnp.float32),
    mesh=plsc.VectorSubcoreMesh(core_axis_name="c", subcore_axis_name="s"),
    scratch_types=[
        pltpu.VMEM((CHH, CE), jnp.int32),
        pltpu.VMEM((CHH, CE), jnp.int32),
        (pltpu.VMEM((CE, D), jnp.float32), pltpu.VMEM((CE, D), jnp.float32)),
        pltpu.VMEM_SHARED((ACCN, D), jnp.float32),
        (pltpu.SemaphoreType.DMA, pltpu.SemaphoreType.DMA),
    ],
)(_edge_body)


def _pool_body(h_ref, gid_ref, gw_ref, gb_ref, fw_ref, fb_ref, out_ref):
    h = h_ref[...]
    gate = jnp.sum(h * gw_ref[...], axis=1, keepdims=True) + gb_ref[0, 0]
    onehot_b = gid_ref[...] == jax.lax.broadcasted_iota(jnp.int32, (1, G), 1)
    one = onehot_b.astype(jnp.float32)
    gmax = jnp.max(jnp.where(onehot_b, gate, -1e30), axis=0, keepdims=True)
    ge = jnp.exp(gate - jnp.sum(one * gmax, axis=1, keepdims=True))
    denom = jnp.sum(one * ge, axis=0, keepdims=True)
    denom_n = jnp.sum(one * denom, axis=1, keepdims=True)
    wh = (ge / jnp.maximum(denom_n, 1e-12)) * h
    hg = jax.lax.dot_general(one, wh, (((0,), (0,)), ((), ())),
                             precision=_PREC,
                             preferred_element_type=jnp.float32)
    out_ref[...] = _matmul(hg, fw_ref[...]) + fb_ref[...]


_pool_call = pl.pallas_call(
    _pool_body,
    in_specs=[
        pl.BlockSpec((N, D), lambda: (0, 0)),
        pl.BlockSpec((N, 1), lambda: (0, 0)),
        pl.BlockSpec((1, D), lambda: (0, 0)),
        pl.BlockSpec((1, 1), lambda: (0, 0)),
        pl.BlockSpec((NCLS, D), lambda: (0, 0)),
        pl.BlockSpec((1, NCLS), lambda: (0, 0)),
    ],
    out_specs=pl.BlockSpec((G, NCLS), lambda: (0, 0)),
    out_shape=jax.ShapeDtypeStruct((G, NCLS), jnp.float32),
)


def kernel(feat, edge_index, etypes, graph_ids, W1, b1, gru1_wih, gru1_whh,
           gru1_bih, gru1_bhh, W2, b2, gru2_wih, gru2_whh, gru2_bih, gru2_bhh,
           gate_w, gate_b, fc_w, fc_b):
    src = edge_index[0].astype(jnp.int32)
    dst = edge_index[1].astype(jnp.int32)
    nw = NC * NS
    gidx = (etypes.astype(jnp.int32) * N + src).reshape(nw, NSTAGE, CHH, CE)
    didx = dst.reshape(nw, NSTAGE, CHH, CE)
    zeros = jnp.zeros((ACCN, D), jnp.float32)
    gid2 = graph_ids.astype(jnp.int32).reshape(N, 1)
    bih1 = gru1_bih.reshape(1, 3 * D)
    bhh1 = gru1_bhh.reshape(1, 3 * D)
    bih2 = gru2_bih.reshape(1, 3 * D)
    bhh2 = gru2_bhh.reshape(1, 3 * D)
    gb2 = gate_b.reshape(1, 1)
    fb2 = fc_b.reshape(1, NCLS)

    h = feat
    m = _mm_call(h, W1, b1)
    for layer in range(2):
        wih, whh, bih, bhh = ((gru1_wih, gru1_whh, bih1, bhh1) if layer == 0
                              else (gru2_wih, gru2_whh, bih2, bhh2))
        for step in range(NSTEPS):
            ap = _edge_call(m.reshape(NE * N, D), gidx, didx, zeros)
            last = layer == 1 and step == NSTEPS - 1
            boundary = layer == 0 and step == NSTEPS - 1
            if last:
                (h,) = _gru_last_call(ap, h, wih, whh, bih, bhh, W2, b2)
            elif boundary:
                h, m = _gru_m_relu_call(ap, h, wih, whh, bih, bhh, W2, b2)
            else:
                Wc, bc = (W1, b1) if layer == 0 else (W2, b2)
                h, m = _gru_m_call(ap, h, wih, whh, bih, bhh, Wc, bc)
    return _pool_call(h, gid2, gate_w, gb2, fc_w, fb2)


# speedup vs baseline: 3.1056x; 1.0231x over previous
import functools

import jax
import jax.numpy as jnp
from jax import lax
from jax.experimental import pallas as pl
from jax.experimental.pallas import tpu as pltpu
from jax.experimental.pallas import tpu_sc as plsc

N = 10000
E = 320000
D = 128
NE = 4
NSTEPS = 3
G = 128
NCLS = 10

NC = 2
NS = 16
CE = 100
CH = E // (NC * NS * CE)
NSTAGE = 2
CHH = CH // NSTAGE
ACCN = N
RPT = 624

_DN = (((1,), (1,)), ((), ()))
_PREC = jax.lax.Precision.DEFAULT

NB = 5
BN = N // NB


def _matmul(x, w):
    return jax.lax.dot_general(x, w, _DN, precision=_PREC,
                               preferred_element_type=jnp.float32)


def _gru(a, h, wih, whh, bih, bhh):
    gi = _matmul(a, wih) + bih
    gh = _matmul(h, whh) + bhh
    r = jax.nn.sigmoid(gi[:, :D] + gh[:, :D])
    z = jax.nn.sigmoid(gi[:, D:2 * D] + gh[:, D:2 * D])
    n = jnp.tanh(gi[:, 2 * D:] + r * gh[:, 2 * D:])
    return (1.0 - z) * n + z * h


def _mm_body(h_ref, w_ref, b_ref, m_ref):
    h = h_ref[...]
    for e in range(NE):
        m_ref[e] = _matmul(h, w_ref[e]) + b_ref[e][None, :]


_mm_call = pl.pallas_call(
    _mm_body,
    grid=(NB,),
    in_specs=[
        pl.BlockSpec((BN, D), lambda i: (i, 0)),
        pl.BlockSpec((NE, D, D), lambda i: (0, 0, 0)),
        pl.BlockSpec((NE, D), lambda i: (0, 0)),
    ],
    out_specs=pl.BlockSpec((NE, BN, D), lambda i: (0, i, 0)),
    out_shape=jax.ShapeDtypeStruct((NE, N, D), jnp.float32),
)


def _gru_body(ap_ref, h_ref, wih_ref, whh_ref, bih_ref, bhh_ref, w_ref, b_ref,
              h_out, m_out, *, relu, emit_m):
    a = ap_ref[0] + ap_ref[1]
    hn = _gru(a, h_ref[...], wih_ref[...], whh_ref[...], bih_ref[...],
              bhh_ref[...])
    if relu:
        hn = jnp.maximum(hn, 0.0)
    h_out[...] = hn
    if emit_m:
        for e in range(NE):
            m_out[e] = _matmul(hn, w_ref[e]) + b_ref[e][None, :]


def _make_gru_call(relu, emit_m):
    out_shape = [jax.ShapeDtypeStruct((N, D), jnp.float32)]
    out_specs = [pl.BlockSpec((BN, D), lambda i: (i, 0))]
    if emit_m:
        out_shape.append(jax.ShapeDtypeStruct((NE, N, D), jnp.float32))
        out_specs.append(pl.BlockSpec((NE, BN, D), lambda i: (0, i, 0)))

    def body(ap, h, wih, whh, bih, bhh, w, b, h_out, *maybe_m):
        _gru_body(ap, h, wih, whh, bih, bhh, w, b, h_out,
                  maybe_m[0] if emit_m else None, relu=relu, emit_m=emit_m)

    return pl.pallas_call(
        body,
        grid=(NB,),
        in_specs=[
            pl.BlockSpec((NC, BN, D), lambda i: (0, i, 0)),
            pl.BlockSpec((BN, D), lambda i: (i, 0)),
            pl.BlockSpec((3 * D, D), lambda i: (0, 0)),
            pl.BlockSpec((3 * D, D), lambda i: (0, 0)),
            pl.BlockSpec((1, 3 * D), lambda i: (0, 0)),
            pl.BlockSpec((1, 3 * D), lambda i: (0, 0)),
            pl.BlockSpec((NE, D, D), lambda i: (0, 0, 0)),
            pl.BlockSpec((NE, D), lambda i: (0, 0)),
        ],
        out_specs=out_specs,
        out_shape=out_shape,
    )


_gru_m_call = _make_gru_call(relu=False, emit_m=True)
_gru_m_relu_call = _make_gru_call(relu=True, emit_m=True)
_gru_last_call = _make_gru_call(relu=True, emit_m=False)


def _edge_body(m_hbm, g_hbm, d_hbm, z_hbm, out_hbm, gv, dv, rows, acc, sem,
               zsem):
    c = lax.axis_index("c")
    s = lax.axis_index("s")
    wid = c * NS + s
    tail0 = NS * RPT
    rows0, rows1 = rows
    sem0, sem1 = sem
    zero = pltpu.async_copy(z_hbm.at[pl.ds(s * RPT, RPT)],
                            acc.at[pl.ds(s * RPT, RPT)], zsem)
    pltpu.sync_copy(g_hbm.at[wid, 0], gv)
    pltpu.sync_copy(d_hbm.at[wid, 0], dv)
    pltpu.async_copy(m_hbm.at[gv.at[0]], rows0, sem0)
    zero.wait()

    @pl.when(s == NS - 1)
    def _():
        pltpu.sync_copy(z_hbm.at[pl.ds(tail0, ACCN - tail0)],
                        acc.at[pl.ds(tail0, ACCN - tail0)])

    plsc.subcore_barrier()

    for stage in range(NSTAGE):
        if stage > 0:
            pltpu.sync_copy(g_hbm.at[wid, stage], gv)
            pltpu.sync_copy(d_hbm.at[wid, stage], dv)
            pltpu.async_copy(m_hbm.at[gv.at[0]], rows0, sem0)

        def chunk(i, carry):
            j = 2 * i
            pltpu.async_copy(m_hbm.at[gv.at[j + 1]], rows1, sem1)
            pltpu.make_async_copy(m_hbm.at[gv.at[j]], rows0, sem0).wait()
            pltpu.sync_copy(rows0, acc.at[dv.at[j]], add=True)

            @pl.when(j + 2 < CHH)
            def _():
                pltpu.async_copy(m_hbm.at[gv.at[j + 2]], rows0, sem0)

            pltpu.make_async_copy(m_hbm.at[gv.at[j + 1]], rows1, sem1).wait()
            pltpu.sync_copy(rows1, acc.at[dv.at[j + 1]], add=True)
            return carry

        lax.fori_loop(0, CHH // 2, chunk, 0)
    plsc.subcore_barrier()
    pltpu.sync_copy(acc.at[pl.ds(s * RPT, RPT)],
                    out_hbm.at[c, pl.ds(s * RPT, RPT)])

    @pl.when(s == NS - 1)
    def _():
        pltpu.sync_copy(acc.at[pl.ds(tail0, N - tail0)],
                        out_hbm.at[c, pl.ds(tail0, N - tail0)])


_edge_call = functools.partial(
    pl.kernel,
    out_type=jax.ShapeDtypeStruct((NC, N, D), jnp.float32),
    mesh=plsc.VectorSubcoreMesh(core_axis_name="c", subcore_axis_name="s"),
    scratch_types=[
        pltpu.VMEM((CHH, CE), jnp.int32),
        pltpu.VMEM((CHH, CE), jnp.int32),
        (pltpu.VMEM((CE, D), jnp.float32), pltpu.VMEM((CE, D), jnp.float32)),
        pltpu.VMEM_SHARED((ACCN, D), jnp.float32),
        (pltpu.SemaphoreType.DMA, pltpu.SemaphoreType.DMA),
        pltpu.SemaphoreType.DMA,
    ],
)(_edge_body)


def _pool_body(h_ref, gid_ref, gw_ref, gb_ref, fw_ref, fb_ref, out_ref):
    h = h_ref[...]
    gate = jnp.sum(h * gw_ref[...], axis=1, keepdims=True) + gb_ref[0, 0]
    onehot_b = gid_ref[...] == jax.lax.broadcasted_iota(jnp.int32, (1, G), 1)
    one = onehot_b.astype(jnp.float32)
    gmax = jnp.max(jnp.where(onehot_b, gate, -1e30), axis=0, keepdims=True)
    ge = jnp.exp(gate - jnp.sum(one * gmax, axis=1, keepdims=True))
    denom = jnp.sum(one * ge, axis=0, keepdims=True)
    denom_n = jnp.sum(one * denom, axis=1, keepdims=True)
    wh = (ge / jnp.maximum(denom_n, 1e-12)) * h
    hg = jax.lax.dot_general(one, wh, (((0,), (0,)), ((), ())),
                             precision=_PREC,
                             preferred_element_type=jnp.float32)
    out_ref[...] = _matmul(hg, fw_ref[...]) + fb_ref[...]


_pool_call = pl.pallas_call(
    _pool_body,
    in_specs=[
        pl.BlockSpec((N, D), lambda: (0, 0)),
        pl.BlockSpec((N, 1), lambda: (0, 0)),
        pl.BlockSpec((1, D), lambda: (0, 0)),
        pl.BlockSpec((1, 1), lambda: (0, 0)),
        pl.BlockSpec((NCLS, D), lambda: (0, 0)),
        pl.BlockSpec((1, NCLS), lambda: (0, 0)),
    ],
    out_specs=pl.BlockSpec((G, NCLS), lambda: (0, 0)),
    out_shape=jax.ShapeDtypeStruct((G, NCLS), jnp.float32),
)


def kernel(feat, edge_index, etypes, graph_ids, W1, b1, gru1_wih, gru1_whh,
           gru1_bih, gru1_bhh, W2, b2, gru2_wih, gru2_whh, gru2_bih, gru2_bhh,
           gate_w, gate_b, fc_w, fc_b):
    src = edge_index[0].astype(jnp.int32)
    dst = edge_index[1].astype(jnp.int32)
    nw = NC * NS
    gidx = (etypes.astype(jnp.int32) * N + src).reshape(nw, NSTAGE, CHH, CE)
    didx = dst.reshape(nw, NSTAGE, CHH, CE)
    zeros = jnp.zeros((ACCN, D), jnp.float32)
    gid2 = graph_ids.astype(jnp.int32).reshape(N, 1)
    bih1 = gru1_bih.reshape(1, 3 * D)
    bhh1 = gru1_bhh.reshape(1, 3 * D)
    bih2 = gru2_bih.reshape(1, 3 * D)
    bhh2 = gru2_bhh.reshape(1, 3 * D)
    gb2 = gate_b.reshape(1, 1)
    fb2 = fc_b.reshape(1, NCLS)

    h = feat
    m = _mm_call(h, W1, b1)
    for layer in range(2):
        wih, whh, bih, bhh = ((gru1_wih, gru1_whh, bih1, bhh1) if layer == 0
                              else (gru2_wih, gru2_whh, bih2, bhh2))
        for step in range(NSTEPS):
            ap = _edge_call(m.reshape(NE * N, D), gidx, didx, zeros)
            last = layer == 1 and step == NSTEPS - 1
            boundary = layer == 0 and step == NSTEPS - 1
            if last:
                (h,) = _gru_last_call(ap, h, wih, whh, bih, bhh, W2, b2)
            elif boundary:
                h, m = _gru_m_relu_call(ap, h, wih, whh, bih, bhh, W2, b2)
            else:
                Wc, bc = (W1, b1) if layer == 0 else (W2, b2)
                h, m = _gru_m_call(ap, h, wih, whh, bih, bhh, Wc, bc)
    return _pool_call(h, gid2, gate_w, gb2, fc_w, fb2)
